# baseline (speedup 1.0000x reference)
import math

import jax
import jax.numpy as jnp
from jax import lax
from jax.experimental import pallas as pl
from jax.experimental.pallas import tpu as pltpu

_A = math.exp(-1.0 / (2.0 * 0.8 * 0.8))
_G0 = _A / (1.0 + 2.0 * _A)
_G1 = 1.0 / (1.0 + 2.0 * _A)

_BH = 512
_IN_ROWS = 520


def _shift_lr(a):
    left = jnp.concatenate([a[:, 1:2], a[:, :-1]], axis=1)
    right = jnp.concatenate([a[:, 1:], a[:, -2:-1]], axis=1)
    return left, right


def _gray(a):
    return 0.299 * a[0] + 0.587 * a[1] + 0.114 * a[2]


def _edge_compute(ga, at_top):
    if at_top:
        ext = jnp.concatenate([ga[2:3], ga[1:2], ga], axis=0)
    else:
        ext = jnp.concatenate([ga, ga[-2:-1], ga[-3:-2]], axis=0)

    l, r = _shift_lr(ext)
    tmp = _G1 * ext + _G0 * (l + r)
    bl = _G1 * tmp[1:-1] + _G0 * (tmp[:-2] + tmp[2:])

    l2, r2 = _shift_lr(bl)
    dx = r2 - l2
    sx = 3.0 * (dx[:-2] + dx[2:]) + 10.0 * dx[1:-1]
    sh = 3.0 * (l2 + r2) + 10.0 * bl
    sy = sh[2:] - sh[:-2]

    return jnp.sqrt(sx * sx + sy * sy)


def _edge_body(x_ref, o_ref, in_buf, out_buf, in_sem, out_sem):
    p = pl.program_id(0)
    n_batch = x_ref.shape[0] // pl.num_programs(0)
    H = x_ref.shape[2]
    b0 = n_batch * p

    def in_copy(b, i, slot):
        r0 = i * (H - _IN_ROWS)
        return pltpu.make_async_copy(
            x_ref.at[b, :, pl.ds(r0, _IN_ROWS), :],
            in_buf.at[slot],
            in_sem.at[slot],
        )

    def out_copy(b, i, slot):
        return pltpu.make_async_copy(
            out_buf.at[slot],
            o_ref.at[b, :, pl.ds(i * _BH, _BH), :],
            out_sem.at[slot],
        )

    in_copy(b0, 0, 0).start()
    in_copy(b0, 1, 1).start()

    def step(k, carry):
        b = b0 + k
        for i in (0, 1):
            in_copy(b, i, i).wait()

            @pl.when(k >= 1)
            def _():
                out_copy(b - 1, i, i).wait()

            if i == 0:
                ga = _gray(in_buf[i, :, 0:_BH + 2, :])
                out_buf[i] = jnp.broadcast_to(
                    _edge_compute(ga, at_top=True)[None], out_buf.shape[1:]
                )
            else:
                ga = _gray(in_buf[i, :, _IN_ROWS - _BH - 2:_IN_ROWS, :])
                out_buf[i] = jnp.broadcast_to(
                    _edge_compute(ga, at_top=False)[None], out_buf.shape[1:]
                )

            out_copy(b, i, i).start()

            @pl.when(k + 1 < n_batch)
            def _():
                in_copy(b + 1, i, i).start()
        return carry

    lax.fori_loop(0, n_batch, step, 0)

    b_last = b0 + n_batch - 1
    out_copy(b_last, 0, 0).wait()
    out_copy(b_last, 1, 1).wait()


def kernel(x):
    B, C, H, W = x.shape
    assert H == 2 * _BH

    return pl.pallas_call(
        _edge_body,
        out_shape=jax.ShapeDtypeStruct((B, C, H, W), x.dtype),
        grid=(2,),
        in_specs=[pl.BlockSpec(memory_space=pl.ANY)],
        out_specs=pl.BlockSpec(memory_space=pl.ANY),
        scratch_shapes=[
            pltpu.VMEM((2, C, _IN_ROWS, W), jnp.float32),
            pltpu.VMEM((2, C, _BH, W), jnp.float32),
            pltpu.SemaphoreType.DMA((2,)),
            pltpu.SemaphoreType.DMA((2,)),
        ],
        compiler_params=pltpu.CompilerParams(
            dimension_semantics=("parallel",),
            vmem_limit_bytes=56 * 1024 * 1024,
        ),
        name="edge_detect_manual",
    )(x)


# speedup vs baseline: 1.0412x; 1.0412x over previous
import math

import jax
import jax.numpy as jnp
from jax.experimental import pallas as pl
from jax.experimental.pallas import tpu as pltpu

_A = math.exp(-1.0 / (2.0 * 0.8 * 0.8))
_G0 = _A / (1.0 + 2.0 * _A)
_G1 = 1.0 / (1.0 + 2.0 * _A)

_BH = 512
_HALO = 8


def _shift_lr(a):
    left = jnp.concatenate([a[:, 1:2], a[:, :-1]], axis=1)
    right = jnp.concatenate([a[:, 1:], a[:, -2:-1]], axis=1)
    return left, right


def _gray(a):
    return 0.299 * a[0] + 0.587 * a[1] + 0.114 * a[2]


def _edge_body(xm_ref, xh_ref, o_ref):
    i = pl.program_id(1)
    n = pl.num_programs(1)
    bh = _BH

    gm = _gray(xm_ref[0])
    gt = _gray(xh_ref[0, :, _HALO - 2:_HALO])
    gb = _gray(xh_ref[0, :, 0:2])

    top_refl = jnp.concatenate([gm[2:3], gm[1:2]], axis=0)
    bot_refl = jnp.concatenate([gm[bh - 2:bh - 1], gm[bh - 3:bh - 2]], axis=0)
    top = jnp.where(i == 0, top_refl, gt)
    bot = jnp.where(i == n - 1, bot_refl, gb)
    g = jnp.concatenate([top, gm, bot], axis=0)

    l, r = _shift_lr(g)
    tmp = _G1 * g + _G0 * (l + r)
    bl = _G1 * tmp[1:-1] + _G0 * (tmp[:-2] + tmp[2:])

    l2, r2 = _shift_lr(bl)
    dx = r2 - l2
    sx = 3.0 * (dx[:-2] + dx[2:]) + 10.0 * dx[1:-1]
    sh = 3.0 * (l2 + r2) + 10.0 * bl
    sy = sh[2:] - sh[:-2]

    mag = jnp.sqrt(sx * sx + sy * sy)
    o_ref[0] = jnp.broadcast_to(mag[None], (3, bh, mag.shape[-1]))


def kernel(x):
    B, C, H, W = x.shape
    bh = _BH
    n = H // bh
    hb = _HALO

    assert n == 2, "kernel assumes two row-blocks per image"

    grid = (B, n)
    main_spec = pl.BlockSpec((1, C, bh, W), lambda b, i: (b, 0, i, 0))
    halo_spec = pl.BlockSpec(
        (1, C, hb, W),
        lambda b, i: (b, 0, bh // hb - i, 0),
    )
    out_spec = pl.BlockSpec((1, C, bh, W), lambda b, i: (b, 0, i, 0))

    return pl.pallas_call(
        _edge_body,
        out_shape=jax.ShapeDtypeStruct((B, C, H, W), x.dtype),
        grid=grid,
        in_specs=[main_spec, halo_spec],
        out_specs=out_spec,
        compiler_params=pltpu.CompilerParams(
            dimension_semantics=("parallel", "arbitrary"),
            vmem_limit_bytes=56 * 1024 * 1024,
        ),
        name="edge_detect_fused",
    )(x, x)


# speedup vs baseline: 1.0415x; 1.0003x over previous
import math

import jax
import jax.numpy as jnp
from jax.experimental import pallas as pl
from jax.experimental.pallas import tpu as pltpu

_A = math.exp(-1.0 / (2.0 * 0.8 * 0.8))
_G0 = _A / (1.0 + 2.0 * _A)
_G1 = 1.0 / (1.0 + 2.0 * _A)

_BH = 512
_HALO = 8


def _shift_lr(a):
    left = jnp.concatenate([a[:, 1:2], a[:, :-1]], axis=1)
    right = jnp.concatenate([a[:, 1:], a[:, -2:-1]], axis=1)
    return left, right


def _gray(a):
    return 0.299 * a[0] + 0.587 * a[1] + 0.114 * a[2]


def _edge_body(xm_ref, xh_ref, o_ref):
    i = pl.program_id(1)
    n = pl.num_programs(1)
    bh = _BH

    gm = _gray(xm_ref[0])
    gt = _gray(xh_ref[0, :, _HALO - 2:_HALO])
    gb = _gray(xh_ref[0, :, 0:2])

    top_refl = jnp.concatenate([gm[2:3], gm[1:2]], axis=0)
    bot_refl = jnp.concatenate([gm[bh - 2:bh - 1], gm[bh - 3:bh - 2]], axis=0)
    top = jnp.where(i == 0, top_refl, gt)
    bot = jnp.where(i == n - 1, bot_refl, gb)
    g = jnp.concatenate([top, gm, bot], axis=0)

    l, r = _shift_lr(g)
    tmp = _G1 * g + _G0 * (l + r)
    bl = _G1 * tmp[1:-1] + _G0 * (tmp[:-2] + tmp[2:])

    l2, r2 = _shift_lr(bl)
    dx = r2 - l2
    sx = 3.0 * (dx[:-2] + dx[2:]) + 10.0 * dx[1:-1]
    sh = 3.0 * (l2 + r2) + 10.0 * bl
    sy = sh[2:] - sh[:-2]

    mag = jnp.sqrt(sx * sx + sy * sy)
    o_ref[0] = jnp.broadcast_to(mag[None], (3, bh, mag.shape[-1]))


def kernel(x):
    B, C, H, W = x.shape
    bh = _BH
    n = H // bh
    hb = _HALO

    assert n == 2, "kernel assumes two row-blocks per image"

    grid = (B, n)
    main_spec = pl.BlockSpec((1, C, bh, W), lambda b, i: (b, 0, i, 0))
    halo_spec = pl.BlockSpec(
        (1, C, hb, W),
        lambda b, i: (b, 0, bh // hb - i, 0),
    )
    out_spec = pl.BlockSpec((1, C, bh, W), lambda b, i: (b, 0, i, 0))

    return pl.pallas_call(
        _edge_body,
        out_shape=jax.ShapeDtypeStruct((B, C, H, W), x.dtype),
        grid=grid,
        in_specs=[main_spec, halo_spec],
        out_specs=out_spec,
        compiler_params=pltpu.CompilerParams(
            dimension_semantics=("parallel", "parallel"),
            vmem_limit_bytes=56 * 1024 * 1024,
        ),
        name="edge_detect_fused",
    )(x, x)
